# baseline (speedup 1.0000x reference)
import functools

import jax
import jax.numpy as jnp
from jax import lax
from jax.experimental import pallas as pl
from jax.experimental.pallas import tpu as pltpu
from jax.experimental.pallas import tpu_sc as plsc

N = 10000
E = 320000
ND = 128
ED = 4
HD = 128
TD = 1
ALPHA = 0.1
TOL = 1e-05
MAX_ITER = 20

NC = 2
NS = 16
NW = NC * NS
CHUNK = 128
NCH = 80
NPASS = 10
P = NCH // NPASS
E_PAD = NW * NCH * CHUNK
W = 64
AGG_ROWS = 10112
ZROWS = AGG_ROWS // NS

M = AGG_ROWS
BM = 2528
GRID = M // BM

E_PAD_T = 328000
BME = 4000

F32 = jnp.float32
I32 = jnp.int32


def _make_seg(table_rows):
    mesh = plsc.VectorSubcoreMesh(core_axis_name="c", subcore_axis_name="s")

    @functools.partial(
        pl.kernel,
        out_type=jax.ShapeDtypeStruct((NC, AGG_ROWS, HD), F32),
        mesh=mesh,
        scratch_types=[
            pltpu.VMEM((2, P, CHUNK), I32),
            pltpu.VMEM((2, P, CHUNK), I32),
            pltpu.VMEM((2, CHUNK, HD), F32),
            pltpu.VMEM((W, HD), F32),
            pltpu.VMEM((16,), I32),
            pltpu.VMEM_SHARED((AGG_ROWS, HD), F32),
            pltpu.SemaphoreType.DMA,
            pltpu.SemaphoreType.DMA,
            pltpu.SemaphoreType.DMA,
            pltpu.SemaphoreType.DMA,
        ],
    )
    def seg(table, gidx3, didx3, zeros, out,
            gstage, dstage, rows, acc, fidx, aggsh, g0, g1, i0, i1):
        cid = lax.axis_index("c")
        sid = lax.axis_index("s")
        wid = sid * NC + cid

        pltpu.sync_copy(zeros, aggsh.at[pl.ds(sid * ZROWS, ZROWS)])
        zv = jnp.zeros((16,), F32)

        def zb(r, c):
            for p8 in range(8):
                acc.at[r][pl.ds(p8 * 16, 16)] = zv
            return c

        lax.fori_loop(0, W, zb, 0)
        plsc.subcore_barrier()

        pltpu.sync_copy(gidx3.at[wid, pl.ds(0, P)], gstage.at[0])
        pltpu.sync_copy(didx3.at[wid, pl.ds(0, P)], dstage.at[0])
        pltpu.async_copy(table.at[gstage.at[0, 0]], rows.at[0], g0)

        def accumulate(b, par, jj):
            dref = dstage.at[par, jj]
            d0 = dref[pl.ds(0, 16)][0]
            dlast = dref[pl.ds(CHUNK - 16, 16)][15]
            span = dlast - d0

            @pl.when(span < W)
            def _fast():
                for g in range(CHUNK // 16):
                    dv = dref[pl.ds(g * 16, 16)]
                    offv = dv - d0
                    offs = [offv[l] for l in range(16)]
                    for l in range(16):
                        e = g * 16 + l
                        for p8 in range(8):
                            v = rows[b, e, pl.ds(p8 * 16, 16)]
                            plsc.addupdate(acc.at[offs[l], pl.ds(p8 * 16, 16)], v)

                def fl(f, c):
                    fidx[...] = lax.iota(I32, 16) + d0 + 16 * f
                    pltpu.sync_copy(acc.at[pl.ds(f * 16, 16)],
                                    aggsh.at[fidx], add=True)
                    for rr in range(16):
                        r = f * 16 + rr
                        for p8 in range(8):
                            acc.at[r][pl.ds(p8 * 16, 16)] = zv
                    return c

                lax.fori_loop(0, span // 16 + 1, fl, 0)

            @pl.when(span >= W)
            def _slow():
                def sb(g, c):
                    fidx[...] = dref[pl.ds(g * 16, 16)]
                    pltpu.sync_copy(rows.at[b, pl.ds(g * 16, 16)],
                                    aggsh.at[fidx], add=True)
                    return c

                lax.fori_loop(0, CHUNK // 16, sb, 0)

        def pair(k, c):
            j0 = 2 * k
            pp = j0 // P
            jj0 = lax.rem(j0, P)
            jj1 = jj0 + 1
            par = lax.rem(pp, 2)

            pltpu.async_copy(table.at[gstage.at[par, jj1]], rows.at[1], g1)

            @pl.when(jnp.logical_and(jj0 == 0, j0 + P < NCH))
            def _():
                nxt = (pp + 1) * P

                @pl.when(par == 0)
                def _():
                    pltpu.async_copy(gidx3.at[wid, pl.ds(nxt, P)], gstage.at[1], i1)
                    pltpu.async_copy(didx3.at[wid, pl.ds(nxt, P)], dstage.at[1], i1)

                @pl.when(par == 1)
                def _():
                    pltpu.async_copy(gidx3.at[wid, pl.ds(nxt, P)], gstage.at[0], i0)
                    pltpu.async_copy(didx3.at[wid, pl.ds(nxt, P)], dstage.at[0], i0)

            pltpu.make_async_copy(table.at[gstage.at[par, jj0]], rows.at[0], g0).wait()
            accumulate(0, par, jj0)

            @pl.when(jnp.logical_and(jj0 + 2 < P, j0 + 2 < NCH))
            def _():
                pltpu.async_copy(table.at[gstage.at[par, jj0 + 2]], rows.at[0], g0)

            @pl.when(jnp.logical_and(jj0 + 2 == P, j0 + 2 < NCH))
            def _():
                nxt = (pp + 1) * P

                @pl.when(par == 0)
                def _():
                    pltpu.make_async_copy(gidx3.at[wid, pl.ds(nxt, P)], gstage.at[1], i1).wait()
                    pltpu.make_async_copy(didx3.at[wid, pl.ds(nxt, P)], dstage.at[1], i1).wait()
                    pltpu.async_copy(table.at[gstage.at[1, 0]], rows.at[0], g0)

                @pl.when(par == 1)
                def _():
                    pltpu.make_async_copy(gidx3.at[wid, pl.ds(nxt, P)], gstage.at[0], i0).wait()
                    pltpu.make_async_copy(didx3.at[wid, pl.ds(nxt, P)], dstage.at[0], i0).wait()
                    pltpu.async_copy(table.at[gstage.at[0, 0]], rows.at[0], g0)

            pltpu.make_async_copy(table.at[gstage.at[par, jj1]], rows.at[1], g1).wait()
            accumulate(1, par, jj1)
            return c

        lax.fori_loop(0, NCH // 2, pair, 0)
        plsc.subcore_barrier()
        pltpu.sync_copy(aggsh.at[pl.ds(sid * ZROWS, ZROWS)],
                        out.at[cid, pl.ds(sid * ZROWS, ZROWS)])

    return seg


_seg = _make_seg(0)


def _efw_body(ef_ref, w1e_ref, out_ref):
    out_ref[...] = jnp.dot(ef_ref[...], w1e_ref[...],
                           preferred_element_type=F32,
                           precision=lax.Precision.HIGHEST)


def _compute_efw(ef_pad, w1e):
    return pl.pallas_call(
        _efw_body,
        grid=(E_PAD_T // BME,),
        in_specs=[
            pl.BlockSpec((BME, ED), lambda i: (i, 0)),
            pl.BlockSpec((ED, HD), lambda i: (0, 0)),
        ],
        out_specs=pl.BlockSpec((BME, HD), lambda i: (i, 0)),
        out_shape=jax.ShapeDtypeStruct((E_PAD_T, HD), F32),
    )(ef_pad, w1e)


def _c_body(nf, anf0, anf1, aef0, aef1, w1a, w1n, b1, c_ref):
    acc = jnp.dot(nf[...], w1a[...], preferred_element_type=F32,
                  precision=lax.Precision.HIGHEST)
    acc += jnp.dot(anf0[0] + anf1[0], w1n[...], preferred_element_type=F32,
                   precision=lax.Precision.HIGHEST)
    c_ref[...] = acc + aef0[0] + aef1[0] + b1[...]


def _compute_c(nf_pad, aggnf, aggefw, w1a, w1n, b1):
    return pl.pallas_call(
        _c_body,
        grid=(GRID,),
        in_specs=[
            pl.BlockSpec((BM, ND), lambda i: (i, 0)),
            pl.BlockSpec((1, BM, ND), lambda i: (0, i, 0)),
            pl.BlockSpec((1, BM, ND), lambda i: (1, i, 0)),
            pl.BlockSpec((1, BM, HD), lambda i: (0, i, 0)),
            pl.BlockSpec((1, BM, HD), lambda i: (1, i, 0)),
            pl.BlockSpec((ND, HD), lambda i: (0, 0)),
            pl.BlockSpec((ND, HD), lambda i: (0, 0)),
            pl.BlockSpec((1, HD), lambda i: (0, 0)),
        ],
        out_specs=pl.BlockSpec((BM, HD), lambda i: (i, 0)),
        out_shape=jax.ShapeDtypeStruct((M, HD), F32),
    )(nf_pad, aggnf, aggnf, aggefw, aggefw, w1a, w1n, b1)


def _iter_body(a0, a1, c, h, w1h, hn_ref, n2_ref):
    i = pl.program_id(0)
    agg = a0[0] + a1[0]
    hn = jnp.maximum(
        jnp.dot(agg, w1h[...], preferred_element_type=F32,
                precision=lax.Precision.HIGHEST) + c[...], 0.0)
    hnew = (1.0 - ALPHA) * h[...] + ALPHA * hn
    grow = i * BM + lax.broadcasted_iota(I32, (BM, HD), 0)
    hnew = jnp.where(grow < N, hnew, 0.0)
    hn_ref[...] = hnew
    d = h[...] - hnew
    s = jnp.sum(d * d, axis=0)
    row = lax.broadcasted_iota(I32, (1, 8, HD), 1)
    n2_ref[...] = jnp.where(row == 0, s[None, None, :], 0.0)


def _iter_tc(agg, c, h, w1h):
    return pl.pallas_call(
        _iter_body,
        grid=(GRID,),
        in_specs=[
            pl.BlockSpec((1, BM, HD), lambda i: (0, i, 0)),
            pl.BlockSpec((1, BM, HD), lambda i: (1, i, 0)),
            pl.BlockSpec((BM, HD), lambda i: (i, 0)),
            pl.BlockSpec((BM, HD), lambda i: (i, 0)),
            pl.BlockSpec((HD, HD), lambda i: (0, 0)),
        ],
        out_specs=[
            pl.BlockSpec((BM, HD), lambda i: (i, 0)),
            pl.BlockSpec((1, 8, HD), lambda i: (i, 0, 0)),
        ],
        out_shape=[
            jax.ShapeDtypeStruct((M, HD), F32),
            jax.ShapeDtypeStruct((GRID, 8, HD), F32),
        ],
    )(agg, agg, c, h, w1h)


BMF = 1000


def _final_body(a0, a1, c, w1h, wr1, br1, wr2, br2, out_ref):
    agg = a0[0] + a1[0]
    hf = jnp.maximum(
        jnp.dot(agg, w1h[...], preferred_element_type=F32,
                precision=lax.Precision.HIGHEST) + c[...], 0.0)
    hr = jnp.maximum(
        jnp.dot(hf, wr1[...], preferred_element_type=F32,
                precision=lax.Precision.HIGHEST) + br1[...], 0.0)
    out_ref[...] = jnp.dot(hr, wr2[...], preferred_element_type=F32,
                           precision=lax.Precision.HIGHEST) + br2[...]


def _final_tc(agg, c, w1h, wr1, br1, wr2, br2):
    return pl.pallas_call(
        _final_body,
        grid=(N // BMF,),
        in_specs=[
            pl.BlockSpec((1, BMF, HD), lambda i: (0, i, 0)),
            pl.BlockSpec((1, BMF, HD), lambda i: (1, i, 0)),
            pl.BlockSpec((BMF, HD), lambda i: (i, 0)),
            pl.BlockSpec((HD, HD), lambda i: (0, 0)),
            pl.BlockSpec((HD, HD), lambda i: (0, 0)),
            pl.BlockSpec((1, HD), lambda i: (0, 0)),
            pl.BlockSpec((HD, TD), lambda i: (0, 0)),
            pl.BlockSpec((1, TD), lambda i: (0, 0)),
        ],
        out_specs=pl.BlockSpec((BMF, TD), lambda i: (i, 0)),
        out_shape=jax.ShapeDtypeStruct((N, TD), F32),
    )(agg, agg, c, w1h, wr1, br1, wr2, br2)


def kernel(nf, edge_index, ef, W1, b1, Wr1, br1, Wr2, br2):
    perm = jnp.argsort(edge_index[1])
    src = edge_index[0][perm]
    dst = edge_index[1][perm]
    pad = E_PAD - E
    d_last = dst[E - 1]

    def chunked(vals, padval):
        v = jnp.concatenate([vals.astype(I32), jnp.full((pad,), padval, I32)])
        return v.reshape(NCH, NW, CHUNK).transpose(1, 0, 2)

    src3 = chunked(src, N)
    dst3 = jnp.concatenate(
        [dst.astype(I32),
         jnp.broadcast_to(d_last.astype(I32), (pad,))]
    ).reshape(NCH, NW, CHUNK).transpose(1, 0, 2)
    eid3 = chunked(perm, E)

    zeros128 = jnp.zeros((ZROWS, HD), F32)
    nf_pad = jnp.zeros((M, ND), F32).at[:N].set(nf)

    w1a = W1[:ND]
    w1h = W1[ND:ND + HD]
    w1n = W1[ND + HD:ND + HD + ND]
    w1e = W1[ND + HD + ND:]

    aggnf = _seg(nf_pad, src3, dst3, zeros128)
    ef_pad = jnp.zeros((E_PAD_T, ED), F32).at[:E].set(ef)
    efw = _compute_efw(ef_pad, w1e)
    aggefw = _seg(efw, eid3, dst3, zeros128)
    c = _compute_c(nf_pad, aggnf, aggefw, w1a, w1n, b1.reshape(1, HD))

    tol2 = jnp.float32(TOL) * jnp.float32(TOL)

    def cond(state):
        i, _, done = state
        return jnp.logical_and(i < MAX_ITER, jnp.logical_not(done))

    def body(state):
        i, h, _ = state
        agg = _seg(h, src3, dst3, zeros128)
        hnew, n2 = _iter_tc(agg, c, h, w1h)
        done = jnp.sum(n2) < tol2
        h = jnp.where(done, h, hnew)
        return (i + 1, h, done)

    h0 = jnp.zeros((M, HD), F32)
    _, h, _ = lax.while_loop(cond, body, (jnp.int32(0), h0, jnp.array(False)))

    agg = _seg(h, src3, dst3, zeros128)
    return _final_tc(agg, c, w1h, Wr1, br1.reshape(1, HD), Wr2, br2.reshape(1, TD))


# speedup vs baseline: 1.1234x; 1.1234x over previous
import functools

import jax
import jax.numpy as jnp
from jax import lax
from jax.experimental import pallas as pl
from jax.experimental.pallas import tpu as pltpu
from jax.experimental.pallas import tpu_sc as plsc

N = 10000
E = 320000
ND = 128
ED = 4
HD = 128
TD = 1
ALPHA = 0.1
TOL = 1e-05
MAX_ITER = 20

NC = 2
NS = 16
NW = NC * NS
CHUNK = 128
NCH = 80
NPASS = 10
P = NCH // NPASS
E_PAD = NW * NCH * CHUNK
W = 64
AGG_ROWS = 10112
ZROWS = AGG_ROWS // NS

M = AGG_ROWS
BM = 2528
GRID = M // BM

E_PAD_T = 328000
BME = 4000

F32 = jnp.float32
I32 = jnp.int32


def _make_seg(table_rows):
    mesh = plsc.VectorSubcoreMesh(core_axis_name="c", subcore_axis_name="s")

    @functools.partial(
        pl.kernel,
        out_type=jax.ShapeDtypeStruct((NC, AGG_ROWS, HD), F32),
        mesh=mesh,
        scratch_types=[
            pltpu.VMEM((2, P, CHUNK), I32),
            pltpu.VMEM((2, P, CHUNK), I32),
            pltpu.VMEM((2, CHUNK, HD), F32),
            pltpu.VMEM((W, HD), F32),
            pltpu.VMEM((16,), I32),
            pltpu.VMEM_SHARED((AGG_ROWS, HD), F32),
            pltpu.SemaphoreType.DMA,
            pltpu.SemaphoreType.DMA,
            pltpu.SemaphoreType.DMA,
            pltpu.SemaphoreType.DMA,
        ],
    )
    def seg(table, gidx3, didx3, zeros, out,
            gstage, dstage, rows, acc, fidx, aggsh, g0, g1, i0, i1):
        cid = lax.axis_index("c")
        sid = lax.axis_index("s")
        wid = sid * NC + cid

        pltpu.sync_copy(zeros, aggsh.at[pl.ds(sid * ZROWS, ZROWS)])
        zv = jnp.zeros((16,), F32)

        def zb(r, c):
            for p8 in range(8):
                acc.at[r][pl.ds(p8 * 16, 16)] = zv
            return c

        lax.fori_loop(0, W, zb, 0)
        plsc.subcore_barrier()

        pltpu.sync_copy(gidx3.at[wid, pl.ds(0, P)], gstage.at[0])
        pltpu.sync_copy(didx3.at[wid, pl.ds(0, P)], dstage.at[0])
        pltpu.async_copy(table.at[gstage.at[0, 0]], rows.at[0], g0)

        def accumulate(b, par, jj):
            dref = dstage.at[par, jj]
            d0 = dref[pl.ds(0, 16)][0]
            dlast = dref[pl.ds(CHUNK - 16, 16)][15]
            span = dlast - d0

            @pl.when(span < W)
            def _fast():
                def gb(g, c):
                    dv = dref[pl.ds(g * 16, 16)]
                    offv = dv - d0
                    for l in range(16):
                        off = offv[l]
                        e = g * 16 + l
                        for p8 in range(8):
                            v = rows[b, e, pl.ds(p8 * 16, 16)]
                            plsc.addupdate(acc.at[off, pl.ds(p8 * 16, 16)], v)
                    return c

                lax.fori_loop(0, CHUNK // 16, gb, 0)

                def fl(f, c):
                    fidx[...] = lax.iota(I32, 16) + d0 + 16 * f
                    pltpu.sync_copy(acc.at[pl.ds(f * 16, 16)],
                                    aggsh.at[fidx], add=True)
                    for rr in range(16):
                        r = f * 16 + rr
                        for p8 in range(8):
                            acc.at[r][pl.ds(p8 * 16, 16)] = zv
                    return c

                lax.fori_loop(0, span // 16 + 1, fl, 0)

            @pl.when(span >= W)
            def _slow():
                def sb(g, c):
                    fidx[...] = dref[pl.ds(g * 16, 16)]
                    pltpu.sync_copy(rows.at[b, pl.ds(g * 16, 16)],
                                    aggsh.at[fidx], add=True)
                    return c

                lax.fori_loop(0, CHUNK // 16, sb, 0)

        def pair(k, c):
            j0 = 2 * k
            pp = j0 // P
            jj0 = lax.rem(j0, P)
            jj1 = jj0 + 1
            par = lax.rem(pp, 2)

            pltpu.async_copy(table.at[gstage.at[par, jj1]], rows.at[1], g1)

            @pl.when(jnp.logical_and(jj0 == 0, j0 + P < NCH))
            def _():
                nxt = (pp + 1) * P

                @pl.when(par == 0)
                def _():
                    pltpu.async_copy(gidx3.at[wid, pl.ds(nxt, P)], gstage.at[1], i1)
                    pltpu.async_copy(didx3.at[wid, pl.ds(nxt, P)], dstage.at[1], i1)

                @pl.when(par == 1)
                def _():
                    pltpu.async_copy(gidx3.at[wid, pl.ds(nxt, P)], gstage.at[0], i0)
                    pltpu.async_copy(didx3.at[wid, pl.ds(nxt, P)], dstage.at[0], i0)

            pltpu.make_async_copy(table.at[gstage.at[par, jj0]], rows.at[0], g0).wait()
            accumulate(0, par, jj0)

            @pl.when(jnp.logical_and(jj0 + 2 < P, j0 + 2 < NCH))
            def _():
                pltpu.async_copy(table.at[gstage.at[par, jj0 + 2]], rows.at[0], g0)

            @pl.when(jnp.logical_and(jj0 + 2 == P, j0 + 2 < NCH))
            def _():
                nxt = (pp + 1) * P

                @pl.when(par == 0)
                def _():
                    pltpu.make_async_copy(gidx3.at[wid, pl.ds(nxt, P)], gstage.at[1], i1).wait()
                    pltpu.make_async_copy(didx3.at[wid, pl.ds(nxt, P)], dstage.at[1], i1).wait()
                    pltpu.async_copy(table.at[gstage.at[1, 0]], rows.at[0], g0)

                @pl.when(par == 1)
                def _():
                    pltpu.make_async_copy(gidx3.at[wid, pl.ds(nxt, P)], gstage.at[0], i0).wait()
                    pltpu.make_async_copy(didx3.at[wid, pl.ds(nxt, P)], dstage.at[0], i0).wait()
                    pltpu.async_copy(table.at[gstage.at[0, 0]], rows.at[0], g0)

            pltpu.make_async_copy(table.at[gstage.at[par, jj1]], rows.at[1], g1).wait()
            accumulate(1, par, jj1)
            return c

        lax.fori_loop(0, NCH // 2, pair, 0)
        plsc.subcore_barrier()
        pltpu.sync_copy(aggsh.at[pl.ds(sid * ZROWS, ZROWS)],
                        out.at[cid, pl.ds(sid * ZROWS, ZROWS)])

    return seg


_seg = _make_seg(0)


def _efw_body(ef_ref, w1e_ref, out_ref):
    out_ref[...] = jnp.dot(ef_ref[...], w1e_ref[...],
                           preferred_element_type=F32)


def _compute_efw(ef_pad, w1e):
    return pl.pallas_call(
        _efw_body,
        grid=(E_PAD_T // BME,),
        in_specs=[
            pl.BlockSpec((BME, ED), lambda i: (i, 0)),
            pl.BlockSpec((ED, HD), lambda i: (0, 0)),
        ],
        out_specs=pl.BlockSpec((BME, HD), lambda i: (i, 0)),
        out_shape=jax.ShapeDtypeStruct((E_PAD_T, HD), F32),
    )(ef_pad, w1e)


def _c_body(nf, anf0, anf1, aef0, aef1, w1a, w1n, b1, c_ref):
    acc = jnp.dot(nf[...], w1a[...], preferred_element_type=F32)
    acc += jnp.dot(anf0[0] + anf1[0], w1n[...], preferred_element_type=F32)
    c_ref[...] = acc + aef0[0] + aef1[0] + b1[...]


def _compute_c(nf_pad, aggnf, aggefw, w1a, w1n, b1):
    return pl.pallas_call(
        _c_body,
        grid=(GRID,),
        in_specs=[
            pl.BlockSpec((BM, ND), lambda i: (i, 0)),
            pl.BlockSpec((1, BM, ND), lambda i: (0, i, 0)),
            pl.BlockSpec((1, BM, ND), lambda i: (1, i, 0)),
            pl.BlockSpec((1, BM, HD), lambda i: (0, i, 0)),
            pl.BlockSpec((1, BM, HD), lambda i: (1, i, 0)),
            pl.BlockSpec((ND, HD), lambda i: (0, 0)),
            pl.BlockSpec((ND, HD), lambda i: (0, 0)),
            pl.BlockSpec((1, HD), lambda i: (0, 0)),
        ],
        out_specs=pl.BlockSpec((BM, HD), lambda i: (i, 0)),
        out_shape=jax.ShapeDtypeStruct((M, HD), F32),
    )(nf_pad, aggnf, aggnf, aggefw, aggefw, w1a, w1n, b1)


def _iter_body(a0, a1, c, h, w1h, hn_ref, n2_ref):
    i = pl.program_id(0)
    agg = a0[0] + a1[0]
    hn = jnp.maximum(
        jnp.dot(agg, w1h[...], preferred_element_type=F32) + c[...], 0.0)
    hnew = (1.0 - ALPHA) * h[...] + ALPHA * hn
    grow = i * BM + lax.broadcasted_iota(I32, (BM, HD), 0)
    hnew = jnp.where(grow < N, hnew, 0.0)
    hn_ref[...] = hnew
    d = h[...] - hnew
    s = jnp.sum(d * d, axis=0)
    row = lax.broadcasted_iota(I32, (1, 8, HD), 1)
    n2_ref[...] = jnp.where(row == 0, s[None, None, :], 0.0)


def _iter_tc(agg, c, h, w1h):
    return pl.pallas_call(
        _iter_body,
        grid=(GRID,),
        in_specs=[
            pl.BlockSpec((1, BM, HD), lambda i: (0, i, 0)),
            pl.BlockSpec((1, BM, HD), lambda i: (1, i, 0)),
            pl.BlockSpec((BM, HD), lambda i: (i, 0)),
            pl.BlockSpec((BM, HD), lambda i: (i, 0)),
            pl.BlockSpec((HD, HD), lambda i: (0, 0)),
        ],
        out_specs=[
            pl.BlockSpec((BM, HD), lambda i: (i, 0)),
            pl.BlockSpec((1, 8, HD), lambda i: (i, 0, 0)),
        ],
        out_shape=[
            jax.ShapeDtypeStruct((M, HD), F32),
            jax.ShapeDtypeStruct((GRID, 8, HD), F32),
        ],
    )(agg, agg, c, h, w1h)


BMF = 1000


def _final_body(a0, a1, c, w1h, wr1, br1, wr2, br2, out_ref):
    agg = a0[0] + a1[0]
    hf = jnp.maximum(
        jnp.dot(agg, w1h[...], preferred_element_type=F32) + c[...], 0.0)
    hr = jnp.maximum(
        jnp.dot(hf, wr1[...], preferred_element_type=F32) + br1[...], 0.0)
    out_ref[...] = jnp.dot(hr, wr2[...], preferred_element_type=F32) + br2[...]


def _final_tc(agg, c, w1h, wr1, br1, wr2, br2):
    return pl.pallas_call(
        _final_body,
        grid=(N // BMF,),
        in_specs=[
            pl.BlockSpec((1, BMF, HD), lambda i: (0, i, 0)),
            pl.BlockSpec((1, BMF, HD), lambda i: (1, i, 0)),
            pl.BlockSpec((BMF, HD), lambda i: (i, 0)),
            pl.BlockSpec((HD, HD), lambda i: (0, 0)),
            pl.BlockSpec((HD, HD), lambda i: (0, 0)),
            pl.BlockSpec((1, HD), lambda i: (0, 0)),
            pl.BlockSpec((HD, TD), lambda i: (0, 0)),
            pl.BlockSpec((1, TD), lambda i: (0, 0)),
        ],
        out_specs=pl.BlockSpec((BMF, TD), lambda i: (i, 0)),
        out_shape=jax.ShapeDtypeStruct((N, TD), F32),
    )(agg, agg, c, w1h, wr1, br1, wr2, br2)


def kernel(nf, edge_index, ef, W1, b1, Wr1, br1, Wr2, br2):
    perm = jnp.argsort(edge_index[1])
    src = edge_index[0][perm]
    dst = edge_index[1][perm]
    pad = E_PAD - E
    d_last = dst[E - 1]

    def chunked(vals, padval):
        v = jnp.concatenate([vals.astype(I32), jnp.full((pad,), padval, I32)])
        return v.reshape(NCH, NW, CHUNK).transpose(1, 0, 2)

    src3 = chunked(src, N)
    dst3 = jnp.concatenate(
        [dst.astype(I32),
         jnp.broadcast_to(d_last.astype(I32), (pad,))]
    ).reshape(NCH, NW, CHUNK).transpose(1, 0, 2)
    eid3 = chunked(perm, E)

    zeros128 = jnp.zeros((ZROWS, HD), F32)
    nf_pad = jnp.zeros((M, ND), F32).at[:N].set(nf)

    w1a = W1[:ND]
    w1h = W1[ND:ND + HD]
    w1n = W1[ND + HD:ND + HD + ND]
    w1e = W1[ND + HD + ND:]

    aggnf = _seg(nf_pad, src3, dst3, zeros128)
    ef_pad = jnp.zeros((E_PAD_T, ED), F32).at[:E].set(ef)
    efw = _compute_efw(ef_pad, w1e)
    aggefw = _seg(efw, eid3, dst3, zeros128)
    c = _compute_c(nf_pad, aggnf, aggefw, w1a, w1n, b1.reshape(1, HD))

    tol2 = jnp.float32(TOL) * jnp.float32(TOL)

    def cond(state):
        i, _, done = state
        return jnp.logical_and(i < MAX_ITER, jnp.logical_not(done))

    def body(state):
        i, h, _ = state
        agg = _seg(h, src3, dst3, zeros128)
        hnew, n2 = _iter_tc(agg, c, h, w1h)
        done = jnp.sum(n2) < tol2
        h = jnp.where(done, h, hnew)
        return (i + 1, h, done)

    h0 = jnp.zeros((M, HD), F32)
    _, h, _ = lax.while_loop(cond, body, (jnp.int32(0), h0, jnp.array(False)))

    agg = _seg(h, src3, dst3, zeros128)
    return _final_tc(agg, c, w1h, Wr1, br1.reshape(1, HD), Wr2, br2.reshape(1, TD))
